# baseline (speedup 1.0000x reference)
import functools

import jax
import jax.numpy as jnp
from jax import lax
from jax.experimental import pallas as pl
from jax.experimental.pallas import tpu as pltpu
from jax.experimental.pallas import tpu_sc as plsc

_K = 99074
_HSZ = 2048 * 16
_CHROWS = 48
_CHW = _CHROWS * 512
_HALF = 384 * 512
_NCH = _HALF // _CHW
_PASSES = ((20, 11, -1), (10, 10, 20), (0, 10, 10))


def _haar_bits_kernel(x_ref, out_ref):
    v = x_ref[0, 0]
    rp = jax.lax.broadcasted_iota(jnp.int32, (512, 512), 0)
    cp = jax.lax.broadcasted_iota(jnp.int32, (512, 512), 1)
    sign = jnp.where(((rp ^ cp) & 1) == 0, 0.5, -0.5)
    z = v * sign
    t = z + jnp.concatenate([z[:, -1:], z[:, :-1]], axis=1)
    u = t + jnp.concatenate([t[-1:, :], t[:-1, :]], axis=0)
    ue = u.reshape(256, 2, 512)[:, 0, :]
    out_ref[0, 0] = jax.lax.bitcast_convert_type(jnp.abs(ue), jnp.int32)


def _sc_select(bits_hbm, sp_hbm, out_hbm, xh_hbm, msg_hbm,
               buf, hist, msg, betav, spv):
    c = lax.axis_index("c")
    s = lax.axis_index("s")
    b = c * 8 + (s // 2)
    half = s % 2
    lane = lax.iota(jnp.int32, 16)
    w_gen = 1 - (lane & 1)
    w_first = w_gen + jnp.where(lane == 0, 1, 0)

    resid = jnp.int32(_K)
    p_scal = jnp.int32(0)
    p_vec = jnp.zeros((16,), jnp.int32)

    for shift, nbits, tshift in _PASSES:
        nbins = 1 << nbits

        def zero_body(i, _):
            hist[pl.ds(i * 16, 16)] = jnp.zeros((16,), jnp.int32)
            return 0
        lax.fori_loop(0, nbins, zero_body, 0)

        for ch in range(_NCH):
            pltpu.sync_copy(
                bits_hbm.at[pl.ds(b * 2 * _HALF + half * _HALF + ch * _CHW,
                                  _CHW)],
                buf.at[pl.ds(0, _CHW)])
            base_row = half * 384 + ch * _CHROWS

            def vreg_body(i, _):
                v = buf[pl.ds(i * 16, 16)]
                bn = (v >> shift) & (nbins - 1)
                wv = w_gen + jnp.where(((i & 31) == 0) & (lane == 0), 1, 0)
                rfac = 1 + jnp.where((base_row + (i >> 5)) % 256 == 0, 1, 0)
                wv = wv * rfac
                if tshift >= 0:
                    wv = jnp.where((v >> tshift) == p_vec, wv, 0)
                plsc.addupdate_scatter(hist, [(bn << 4) + lane], wv)
                return 0
            lax.fori_loop(0, _CHW // 16, vreg_body, 0)

        @pl.when(half == 1)
        def _():
            pltpu.sync_copy(hist.at[pl.ds(0, nbins * 16)],
                            xh_hbm.at[pl.ds(b * _HSZ, nbins * 16)])
        plsc.subcore_barrier()

        @pl.when(half == 0)
        def _():
            pltpu.sync_copy(xh_hbm.at[pl.ds(b * _HSZ, nbins * 16)],
                            buf.at[pl.ds(0, nbins * 16)])
            def merge_body(i, _):
                blk = hist[pl.ds(i * 16, 16)] + buf[pl.ds(i * 16, 16)]
                hist[pl.ds(i * 16, 16)] = blk
                return 0
            lax.fori_loop(0, nbins, merge_body, 0)

        ngroups = nbins // 16

        def group_body(i, carry):
            g = ngroups - 1 - i
            racc, gstar, above = carry
            acc = jnp.zeros((16,), jnp.int32)
            for k in range(16):
                acc = acc + hist[pl.ds(g * 256 + k * 16, 16)]
            t = jnp.sum(acc)
            found = (gstar < 0) & (racc + t >= resid)
            gstar = jnp.where(found, g, gstar)
            above = jnp.where(found, racc, above)
            return racc + t, gstar, above
        _, gstar, g_above = lax.fori_loop(
            0, ngroups, group_body,
            (jnp.int32(0), jnp.int32(-1), jnp.int32(0)))

        def bin_body(i, carry):
            bb = 15 - i
            bacc, bstar, above = carry
            t = jnp.sum(hist[pl.ds((gstar * 16 + bb) * 16, 16)])
            found = (bstar < 0) & (bacc + t >= resid)
            bstar = jnp.where(found, bb, bstar)
            above = jnp.where(found, bacc, above)
            return bacc + t, bstar, above
        _, bstar, b_above = lax.fori_loop(
            0, 16, bin_body, (g_above, jnp.int32(-1), jnp.int32(0)))

        bin_global = gstar * 16 + bstar
        resid = resid - b_above
        p_new = (p_scal << nbits) | bin_global

        @pl.when(half == 0)
        def _():
            msg[...] = jnp.full((16,), p_new, jnp.int32)
            pltpu.sync_copy(msg, msg_hbm.at[pl.ds(b * 16, 16)])
        plsc.subcore_barrier()
        pltpu.sync_copy(msg_hbm.at[pl.ds(b * 16, 16)], msg)
        p_vec = msg[...]
        p_scal = jnp.sum(jnp.where(lane == 0, p_vec, 0))

    med = lax.bitcast_convert_type(p_vec, jnp.float32)
    std = med / 0.6745
    pltpu.sync_copy(sp_hbm.at[pl.ds(0, 16)], spv.at[pl.ds(0, 16)])
    pltpu.sync_copy(sp_hbm.at[pl.ds(16, 16)], spv.at[pl.ds(16, 16)])
    spa = spv[pl.ds(0, 16)]
    spb = spv[pl.ds(16, 16)]
    g = spa * std + spb
    beta = 1.0 / (g * g)

    @pl.when(half == 0)
    def _():
        betav[...] = beta
        pltpu.sync_copy(betav, out_hbm.at[pl.ds(b * 16, 16)])


def kernel(x, a_k, b_k):
    x = x.astype(jnp.float32)

    bits = pl.pallas_call(
        _haar_bits_kernel,
        grid=(16, 3),
        in_specs=[pl.BlockSpec((1, 1, 512, 512), lambda i, j: (i, j, 0, 0))],
        out_specs=pl.BlockSpec((1, 1, 256, 512), lambda i, j: (i, j, 0, 0)),
        out_shape=jax.ShapeDtypeStruct((16, 3, 256, 512), jnp.int32),
    )(x)

    sp = jnp.broadcast_to(
        jax.nn.softplus(jnp.stack([a_k, b_k]))[:, None], (2, 16)).reshape(32)

    mesh = plsc.VectorSubcoreMesh(core_axis_name="c", subcore_axis_name="s")
    sc = functools.partial(
        pl.kernel, mesh=mesh,
        out_type=(
            jax.ShapeDtypeStruct((256,), jnp.float32),
            jax.ShapeDtypeStruct((16 * _HSZ,), jnp.int32),
            jax.ShapeDtypeStruct((256,), jnp.int32),
        ),
        scratch_types=[
            pltpu.VMEM((_HSZ,), jnp.int32),
            pltpu.VMEM((_HSZ,), jnp.int32),
            pltpu.VMEM((16,), jnp.int32),
            pltpu.VMEM((16,), jnp.float32),
            pltpu.VMEM((32,), jnp.float32),
        ],
        compiler_params=pltpu.CompilerParams(needs_layout_passes=False),
    )(_sc_select)
    out, _, _ = sc(bits.reshape(16 * 768 * 512), sp)
    return out.reshape(16, 16)[:, 0]


# speedup vs baseline: 2.4268x; 2.4268x over previous
import functools

import jax
import jax.numpy as jnp
from jax import lax
from jax.experimental import pallas as pl
from jax.experimental.pallas import tpu as pltpu
from jax.experimental.pallas import tpu_sc as plsc

_K = 99074
_NSUB = 2
_UNROLL = 8
_HSZ = 2048 * 16 * _NSUB
_CHROWS = 24
_CHW = _CHROWS * 512
_HALF = 384 * 512
_NCH = _HALF // _CHW
_PASSES = ((20, 11, -1), (10, 10, 20), (0, 10, 10))


def _haar_bits_kernel(x_ref, out_ref):
    v = x_ref[0, 0]
    rp = jax.lax.broadcasted_iota(jnp.int32, (512, 512), 0)
    cp = jax.lax.broadcasted_iota(jnp.int32, (512, 512), 1)
    sign = jnp.where(((rp ^ cp) & 1) == 0, 0.5, -0.5)
    z = v * sign
    t = z + jnp.concatenate([z[:, -1:], z[:, :-1]], axis=1)
    u = t + jnp.concatenate([t[-1:, :], t[:-1, :]], axis=0)
    ue = u.reshape(256, 2, 512)[:, 0, :]
    out_ref[0, 0] = jax.lax.bitcast_convert_type(jnp.abs(ue), jnp.int32)


def _sc_select(bits_hbm, sp_hbm, out_hbm, xh_hbm, msg_hbm,
               bufa, bufb, part, hist, msg, betav, spv, sema, semb):
    c = lax.axis_index("c")
    s = lax.axis_index("s")
    b = c * 8 + (s // 2)
    half = s % 2
    lane = lax.iota(jnp.int32, 16)
    w_gen = 1 - (lane & 1)

    resid = jnp.int32(_K)
    p_scal = jnp.int32(0)
    p_vec = jnp.zeros((16,), jnp.int32)

    bufs = (bufa, bufb)
    sems = (sema, semb)

    def start(ch):
        i = ch & 1
        return pltpu.async_copy(
            bits_hbm.at[pl.ds(b * 2 * _HALF + half * _HALF + ch * _CHW,
                              _CHW)],
            bufs[i], sems[i])

    for shift, nbits, tshift in _PASSES:
        nbins = 1 << nbits
        hwords = nbins * 16

        @plsc.parallel_loop(0, _NSUB * nbins, step=1, unroll=_UNROLL)
        def _(i):
            hist[pl.ds(i * 16, 16)] = jnp.zeros((16,), jnp.int32)

        pend = [start(0)]
        for ch in range(_NCH):
            pend.pop().wait()
            if ch + 1 < _NCH:
                pend.append(start(ch + 1))
            buf = bufs[ch & 1]
            base_row = half * 384 + ch * _CHROWS

            @plsc.parallel_loop(0, _CHW // 16, step=1, unroll=_UNROLL)
            def _(i):
                v = buf[pl.ds(i * 16, 16)]
                bn = (v >> shift) & (nbins - 1)
                wv = w_gen + jnp.where(((i & 31) == 0) & (lane == 0), 1, 0)
                rfac = 1 + jnp.where((base_row + (i >> 5)) % 256 == 0, 1, 0)
                wv = wv * rfac
                if tshift >= 0:
                    wv = jnp.where((v >> tshift) == p_vec, wv, 0)
                sub = (i & 1) * hwords
                plsc.addupdate_scatter(hist, [sub + (bn << 4) + lane], wv)

        @plsc.parallel_loop(0, nbins, step=1, unroll=_UNROLL)
        def _(i):
            blk = hist[pl.ds(i * 16, 16)] + hist[pl.ds(hwords + i * 16, 16)]
            hist[pl.ds(i * 16, 16)] = blk

        @pl.when(half == 1)
        def _():
            pltpu.sync_copy(hist.at[pl.ds(0, hwords)],
                            xh_hbm.at[pl.ds(b * hwords, hwords)])
        plsc.subcore_barrier()

        @pl.when(half == 0)
        def _():
            pltpu.sync_copy(xh_hbm.at[pl.ds(b * hwords, hwords)],
                            part.at[pl.ds(0, hwords)])

            @plsc.parallel_loop(0, nbins, step=1, unroll=_UNROLL)
            def _(i):
                blk = hist[pl.ds(i * 16, 16)] + part[pl.ds(i * 16, 16)]
                hist[pl.ds(i * 16, 16)] = blk

        ngroups = nbins // 16

        def group_body(i, carry):
            g = ngroups - 1 - i
            racc, gstar, above = carry
            acc = jnp.zeros((16,), jnp.int32)
            for k in range(16):
                acc = acc + hist[pl.ds(g * 256 + k * 16, 16)]
            t = jnp.sum(acc)
            found = (gstar < 0) & (racc + t >= resid)
            gstar = jnp.where(found, g, gstar)
            above = jnp.where(found, racc, above)
            return racc + t, gstar, above
        _, gstar, g_above = lax.fori_loop(
            0, ngroups, group_body,
            (jnp.int32(0), jnp.int32(-1), jnp.int32(0)))

        def bin_body(i, carry):
            bb = 15 - i
            bacc, bstar, above = carry
            t = jnp.sum(hist[pl.ds((gstar * 16 + bb) * 16, 16)])
            found = (bstar < 0) & (bacc + t >= resid)
            bstar = jnp.where(found, bb, bstar)
            above = jnp.where(found, bacc, above)
            return bacc + t, bstar, above
        _, bstar, b_above = lax.fori_loop(
            0, 16, bin_body, (g_above, jnp.int32(-1), jnp.int32(0)))

        bin_global = gstar * 16 + bstar
        resid = resid - b_above
        p_new = (p_scal << nbits) | bin_global

        @pl.when(half == 0)
        def _():
            msg[...] = jnp.full((16,), p_new, jnp.int32)
            pltpu.sync_copy(msg, msg_hbm.at[pl.ds(b * 16, 16)])
        plsc.subcore_barrier()
        pltpu.sync_copy(msg_hbm.at[pl.ds(b * 16, 16)], msg)
        p_vec = msg[...]
        p_scal = jnp.sum(jnp.where(lane == 0, p_vec, 0))

    med = lax.bitcast_convert_type(p_vec, jnp.float32)
    std = med / 0.6745
    pltpu.sync_copy(sp_hbm.at[pl.ds(0, 16)], spv.at[pl.ds(0, 16)])
    pltpu.sync_copy(sp_hbm.at[pl.ds(16, 16)], spv.at[pl.ds(16, 16)])
    spa = spv[pl.ds(0, 16)]
    spb = spv[pl.ds(16, 16)]
    g = spa * std + spb
    beta = 1.0 / (g * g)

    @pl.when(half == 0)
    def _():
        betav[...] = beta
        pltpu.sync_copy(betav, out_hbm.at[pl.ds(b * 16, 16)])


def kernel(x, a_k, b_k):
    x = x.astype(jnp.float32)

    bits = pl.pallas_call(
        _haar_bits_kernel,
        grid=(16, 3),
        in_specs=[pl.BlockSpec((1, 1, 512, 512), lambda i, j: (i, j, 0, 0))],
        out_specs=pl.BlockSpec((1, 1, 256, 512), lambda i, j: (i, j, 0, 0)),
        out_shape=jax.ShapeDtypeStruct((16, 3, 256, 512), jnp.int32),
    )(x)

    sp = jnp.broadcast_to(
        jax.nn.softplus(jnp.stack([a_k, b_k]))[:, None], (2, 16)).reshape(32)

    mesh = plsc.VectorSubcoreMesh(core_axis_name="c", subcore_axis_name="s")
    sc = functools.partial(
        pl.kernel, mesh=mesh,
        out_type=(
            jax.ShapeDtypeStruct((256,), jnp.float32),
            jax.ShapeDtypeStruct((16 * 2048 * 16,), jnp.int32),
            jax.ShapeDtypeStruct((256,), jnp.int32),
        ),
        scratch_types=[
            pltpu.VMEM((_CHW,), jnp.int32),
            pltpu.VMEM((_CHW,), jnp.int32),
            pltpu.VMEM((2048 * 16,), jnp.int32),
            pltpu.VMEM((_HSZ,), jnp.int32),
            pltpu.VMEM((16,), jnp.int32),
            pltpu.VMEM((16,), jnp.float32),
            pltpu.VMEM((32,), jnp.float32),
            pltpu.SemaphoreType.DMA,
            pltpu.SemaphoreType.DMA,
        ],
        compiler_params=pltpu.CompilerParams(needs_layout_passes=False),
    )(_sc_select)
    out, _, _ = sc(bits.reshape(16 * 768 * 512), sp)
    return out.reshape(16, 16)[:, 0]


# speedup vs baseline: 2.6677x; 1.0993x over previous
import functools

import jax
import jax.numpy as jnp
from jax import lax
from jax.experimental import pallas as pl
from jax.experimental.pallas import tpu as pltpu
from jax.experimental.pallas import tpu_sc as plsc

_K = 99074
_NSUB = 1
_UNROLL = 16
_HSZ = 2048 * 16 * _NSUB
_CHROWS = 48
_CHW = _CHROWS * 512
_HALF = 384 * 512
_NCH = _HALF // _CHW
_PASSES = ((20, 11, -1), (10, 10, 20), (0, 10, 10))


def _haar_bits_kernel(x_ref, out_ref):
    v = x_ref[0, 0]
    rp = jax.lax.broadcasted_iota(jnp.int32, (512, 512), 0)
    cp = jax.lax.broadcasted_iota(jnp.int32, (512, 512), 1)
    sign = jnp.where(((rp ^ cp) & 1) == 0, 0.5, -0.5)
    z = v * sign
    t = z + jnp.concatenate([z[:, -1:], z[:, :-1]], axis=1)
    u = t + jnp.concatenate([t[-1:, :], t[:-1, :]], axis=0)
    ue = u.reshape(256, 2, 512)[:, 0, :]
    out_ref[0, 0] = jax.lax.bitcast_convert_type(jnp.abs(ue), jnp.int32)


def _sc_select(bits_hbm, sp_hbm, out_hbm, xh_hbm, msg_hbm,
               bufa, bufb, part, hist, msg, betav, spv, sema, semb):
    c = lax.axis_index("c")
    s = lax.axis_index("s")
    b = c * 8 + (s // 2)
    half = s % 2
    lane = lax.iota(jnp.int32, 16)
    w_gen = 1 - (lane & 1)

    resid = jnp.int32(_K)
    p_scal = jnp.int32(0)
    p_vec = jnp.zeros((16,), jnp.int32)

    bufs = (bufa, bufb)
    sems = (sema, semb)

    def start(ch):
        i = ch & 1
        return pltpu.async_copy(
            bits_hbm.at[pl.ds(b * 2 * _HALF + half * _HALF + ch * _CHW,
                              _CHW)],
            bufs[i], sems[i])

    for shift, nbits, tshift in _PASSES:
        nbins = 1 << nbits
        hwords = nbins * 16

        @plsc.parallel_loop(0, _NSUB * nbins, step=1, unroll=_UNROLL)
        def _(i):
            hist[pl.ds(i * 16, 16)] = jnp.zeros((16,), jnp.int32)

        pend = [start(0)]
        for ch in range(_NCH):
            pend.pop().wait()
            if ch + 1 < _NCH:
                pend.append(start(ch + 1))
            buf = bufs[ch & 1]
            base_row = half * 384 + ch * _CHROWS

            @plsc.parallel_loop(0, _CHW // 16, step=1, unroll=_UNROLL)
            def _(i):
                v = buf[pl.ds(i * 16, 16)]
                bn = (v >> shift) & (nbins - 1)
                wv = w_gen + jnp.where(((i & 31) == 0) & (lane == 0), 1, 0)
                rfac = 1 + jnp.where((base_row + (i >> 5)) % 256 == 0, 1, 0)
                wv = wv * rfac
                if tshift >= 0:
                    wv = jnp.where((v >> tshift) == p_vec, wv, 0)
                plsc.addupdate_scatter(hist, [(bn << 4) + lane], wv)

        @pl.when(half == 1)
        def _():
            pltpu.sync_copy(hist.at[pl.ds(0, hwords)],
                            xh_hbm.at[pl.ds(b * hwords, hwords)])
        plsc.subcore_barrier()

        @pl.when(half == 0)
        def _():
            pltpu.sync_copy(xh_hbm.at[pl.ds(b * hwords, hwords)],
                            part.at[pl.ds(0, hwords)])

            @plsc.parallel_loop(0, nbins, step=1, unroll=_UNROLL)
            def _(i):
                blk = hist[pl.ds(i * 16, 16)] + part[pl.ds(i * 16, 16)]
                hist[pl.ds(i * 16, 16)] = blk

        ngroups = nbins // 16

        def group_body(i, carry):
            g = ngroups - 1 - i
            racc, gstar, above = carry
            acc = jnp.zeros((16,), jnp.int32)
            for k in range(16):
                acc = acc + hist[pl.ds(g * 256 + k * 16, 16)]
            t = jnp.sum(acc)
            found = (gstar < 0) & (racc + t >= resid)
            gstar = jnp.where(found, g, gstar)
            above = jnp.where(found, racc, above)
            return racc + t, gstar, above
        _, gstar, g_above = lax.fori_loop(
            0, ngroups, group_body,
            (jnp.int32(0), jnp.int32(-1), jnp.int32(0)))

        def bin_body(i, carry):
            bb = 15 - i
            bacc, bstar, above = carry
            t = jnp.sum(hist[pl.ds((gstar * 16 + bb) * 16, 16)])
            found = (bstar < 0) & (bacc + t >= resid)
            bstar = jnp.where(found, bb, bstar)
            above = jnp.where(found, bacc, above)
            return bacc + t, bstar, above
        _, bstar, b_above = lax.fori_loop(
            0, 16, bin_body, (g_above, jnp.int32(-1), jnp.int32(0)))

        bin_global = gstar * 16 + bstar
        resid = resid - b_above
        p_new = (p_scal << nbits) | bin_global

        @pl.when(half == 0)
        def _():
            msg[...] = jnp.full((16,), p_new, jnp.int32)
            pltpu.sync_copy(msg, msg_hbm.at[pl.ds(b * 16, 16)])
        plsc.subcore_barrier()
        pltpu.sync_copy(msg_hbm.at[pl.ds(b * 16, 16)], msg)
        p_vec = msg[...]
        p_scal = jnp.sum(jnp.where(lane == 0, p_vec, 0))

    med = lax.bitcast_convert_type(p_vec, jnp.float32)
    std = med / 0.6745
    pltpu.sync_copy(sp_hbm.at[pl.ds(0, 16)], spv.at[pl.ds(0, 16)])
    pltpu.sync_copy(sp_hbm.at[pl.ds(16, 16)], spv.at[pl.ds(16, 16)])
    spa = spv[pl.ds(0, 16)]
    spb = spv[pl.ds(16, 16)]
    g = spa * std + spb
    beta = 1.0 / (g * g)

    @pl.when(half == 0)
    def _():
        betav[...] = beta
        pltpu.sync_copy(betav, out_hbm.at[pl.ds(b * 16, 16)])


def kernel(x, a_k, b_k):
    x = x.astype(jnp.float32)

    bits = pl.pallas_call(
        _haar_bits_kernel,
        grid=(16, 3),
        in_specs=[pl.BlockSpec((1, 1, 512, 512), lambda i, j: (i, j, 0, 0))],
        out_specs=pl.BlockSpec((1, 1, 256, 512), lambda i, j: (i, j, 0, 0)),
        out_shape=jax.ShapeDtypeStruct((16, 3, 256, 512), jnp.int32),
    )(x)

    sp = jnp.broadcast_to(
        jax.nn.softplus(jnp.stack([a_k, b_k]))[:, None], (2, 16)).reshape(32)

    mesh = plsc.VectorSubcoreMesh(core_axis_name="c", subcore_axis_name="s")
    sc = functools.partial(
        pl.kernel, mesh=mesh,
        out_type=(
            jax.ShapeDtypeStruct((256,), jnp.float32),
            jax.ShapeDtypeStruct((16 * 2048 * 16,), jnp.int32),
            jax.ShapeDtypeStruct((256,), jnp.int32),
        ),
        scratch_types=[
            pltpu.VMEM((_CHW,), jnp.int32),
            pltpu.VMEM((_CHW,), jnp.int32),
            pltpu.VMEM((2048 * 16,), jnp.int32),
            pltpu.VMEM((_HSZ,), jnp.int32),
            pltpu.VMEM((16,), jnp.int32),
            pltpu.VMEM((16,), jnp.float32),
            pltpu.VMEM((32,), jnp.float32),
            pltpu.SemaphoreType.DMA,
            pltpu.SemaphoreType.DMA,
        ],
        compiler_params=pltpu.CompilerParams(needs_layout_passes=False),
    )(_sc_select)
    out, _, _ = sc(bits.reshape(16 * 768 * 512), sp)
    return out.reshape(16, 16)[:, 0]


# speedup vs baseline: 3.0061x; 1.1268x over previous
import functools

import jax
import jax.numpy as jnp
from jax import lax
from jax.experimental import pallas as pl
from jax.experimental.pallas import tpu as pltpu
from jax.experimental.pallas import tpu_sc as plsc

_K = 99074
_NSUB = 1
_UNROLL = 16
_HSZ = 2048 * 16 * _NSUB
_CHROWS = 32
_CHW = _CHROWS * 512
_HALF = 384 * 512
_NCH = 384 // _CHROWS
_PASSES = ((20, 11, -1), (10, 10, 20), (0, 10, 10))


def _haar_bits_kernel(x_ref, out_ref):
    v = x_ref[0, 0]
    rp = jax.lax.broadcasted_iota(jnp.int32, (512, 512), 0)
    cp = jax.lax.broadcasted_iota(jnp.int32, (512, 512), 1)
    sign = jnp.where(((rp ^ cp) & 1) == 0, 0.5, -0.5)
    z = v * sign
    t = z + jnp.concatenate([z[:, -1:], z[:, :-1]], axis=1)
    u = t + jnp.concatenate([t[-1:, :], t[:-1, :]], axis=0)
    ue = u.reshape(256, 2, 512)[:, 0, :]
    out_ref[0, 0] = jax.lax.bitcast_convert_type(jnp.abs(ue), jnp.int32)


def _sc_select(bits_hbm, sp_hbm, out_hbm, xh_hbm, msg_hbm,
               bufa, bufb, part, hist, msg, betav, spv, sema, semb):
    c = lax.axis_index("c")
    s = lax.axis_index("s")
    b = c * 8 + (s // 2)
    half = s % 2
    lane = lax.iota(jnp.int32, 16)
    w_gen = 1 - (lane & 1)

    resid = jnp.int32(_K)
    p_scal = jnp.int32(0)
    p_vec = jnp.zeros((16,), jnp.int32)

    bufs = (bufa, bufb)
    sems = (sema, semb)

    def start(ch, h):
        g = h * 384 + ch * _CHROWS
        ci, r0 = divmod(g, 256)
        i = ch & 1
        return pltpu.async_copy(
            bits_hbm.at[b, ci, pl.ds(r0, _CHROWS), :], bufs[i], sems[i])

    for shift, nbits, tshift in _PASSES:
        nbins = 1 << nbits
        hwords = nbins * 16

        @plsc.parallel_loop(0, _NSUB * nbins, step=1, unroll=_UNROLL)
        def _(i):
            hist[pl.ds(i * 16, 16)] = jnp.zeros((16,), jnp.int32)

        pend = [start(0, half)]
        for ch in range(_NCH):
            pend.pop().wait()
            if ch + 1 < _NCH:
                pend.append(start(ch + 1, half))
            buf = bufs[ch & 1]
            base_row = half * 384 + ch * _CHROWS

            @plsc.parallel_loop(0, _CHW // 16, step=1, unroll=_UNROLL)
            def _(i):
                v = buf[i >> 5, pl.ds((i & 31) * 16, 16)]
                bn = (v >> shift) & (nbins - 1)
                wv = w_gen + jnp.where(((i & 31) == 0) & (lane == 0), 1, 0)
                rfac = 1 + jnp.where((base_row + (i >> 5)) % 256 == 0, 1, 0)
                wv = wv * rfac
                if tshift >= 0:
                    wv = jnp.where((v >> tshift) == p_vec, wv, 0)
                plsc.addupdate_scatter(hist, [(bn << 4) + lane], wv)

        @pl.when(half == 1)
        def _():
            pltpu.sync_copy(hist.at[pl.ds(0, hwords)],
                            xh_hbm.at[pl.ds(b * hwords, hwords)])
        plsc.subcore_barrier()

        @pl.when(half == 0)
        def _():
            pltpu.sync_copy(xh_hbm.at[pl.ds(b * hwords, hwords)],
                            part.at[pl.ds(0, hwords)])

            @plsc.parallel_loop(0, nbins, step=1, unroll=_UNROLL)
            def _(i):
                blk = hist[pl.ds(i * 16, 16)] + part[pl.ds(i * 16, 16)]
                hist[pl.ds(i * 16, 16)] = blk

        ngroups = nbins // 16

        def group_body(i, carry):
            g = ngroups - 1 - i
            racc, gstar, above = carry
            acc = jnp.zeros((16,), jnp.int32)
            for k in range(16):
                acc = acc + hist[pl.ds(g * 256 + k * 16, 16)]
            t = jnp.sum(acc)
            found = (gstar < 0) & (racc + t >= resid)
            gstar = jnp.where(found, g, gstar)
            above = jnp.where(found, racc, above)
            return racc + t, gstar, above
        _, gstar, g_above = lax.fori_loop(
            0, ngroups, group_body,
            (jnp.int32(0), jnp.int32(-1), jnp.int32(0)))

        def bin_body(i, carry):
            bb = 15 - i
            bacc, bstar, above = carry
            t = jnp.sum(hist[pl.ds((gstar * 16 + bb) * 16, 16)])
            found = (bstar < 0) & (bacc + t >= resid)
            bstar = jnp.where(found, bb, bstar)
            above = jnp.where(found, bacc, above)
            return bacc + t, bstar, above
        _, bstar, b_above = lax.fori_loop(
            0, 16, bin_body, (g_above, jnp.int32(-1), jnp.int32(0)))

        bin_global = gstar * 16 + bstar
        resid = resid - b_above
        p_new = (p_scal << nbits) | bin_global

        @pl.when(half == 0)
        def _():
            msg[...] = jnp.full((16,), p_new, jnp.int32)
            pltpu.sync_copy(msg, msg_hbm.at[pl.ds(b * 16, 16)])
        plsc.subcore_barrier()
        pltpu.sync_copy(msg_hbm.at[pl.ds(b * 16, 16)], msg)
        p_vec = msg[...]
        p_scal = jnp.sum(jnp.where(lane == 0, p_vec, 0))

    med = lax.bitcast_convert_type(p_vec, jnp.float32)
    std = med / 0.6745
    pltpu.sync_copy(sp_hbm.at[pl.ds(0, 16)], spv.at[pl.ds(0, 16)])
    pltpu.sync_copy(sp_hbm.at[pl.ds(16, 16)], spv.at[pl.ds(16, 16)])
    spa = spv[pl.ds(0, 16)]
    spb = spv[pl.ds(16, 16)]
    g = spa * std + spb
    beta = 1.0 / (g * g)

    @pl.when(half == 0)
    def _():
        betav[...] = beta
        pltpu.sync_copy(betav, out_hbm.at[pl.ds(b * 16, 16)])


def kernel(x, a_k, b_k):
    x = x.astype(jnp.float32)

    bits = pl.pallas_call(
        _haar_bits_kernel,
        grid=(16, 3),
        in_specs=[pl.BlockSpec((1, 1, 512, 512), lambda i, j: (i, j, 0, 0))],
        out_specs=pl.BlockSpec((1, 1, 256, 512), lambda i, j: (i, j, 0, 0)),
        out_shape=jax.ShapeDtypeStruct((16, 3, 256, 512), jnp.int32),
    )(x)

    sp = jnp.broadcast_to(
        jax.nn.softplus(jnp.stack([a_k, b_k]))[:, None], (2, 16)).reshape(32)

    mesh = plsc.VectorSubcoreMesh(core_axis_name="c", subcore_axis_name="s")
    sc = functools.partial(
        pl.kernel, mesh=mesh,
        out_type=(
            jax.ShapeDtypeStruct((256,), jnp.float32),
            jax.ShapeDtypeStruct((16 * 2048 * 16,), jnp.int32),
            jax.ShapeDtypeStruct((256,), jnp.int32),
        ),
        scratch_types=[
            pltpu.VMEM((_CHROWS, 512), jnp.int32),
            pltpu.VMEM((_CHROWS, 512), jnp.int32),
            pltpu.VMEM((2048 * 16,), jnp.int32),
            pltpu.VMEM((_HSZ,), jnp.int32),
            pltpu.VMEM((16,), jnp.int32),
            pltpu.VMEM((16,), jnp.float32),
            pltpu.VMEM((32,), jnp.float32),
            pltpu.SemaphoreType.DMA,
            pltpu.SemaphoreType.DMA,
        ],
        compiler_params=pltpu.CompilerParams(needs_layout_passes=False),
    )(_sc_select)
    out, _, _ = sc(bits, sp)
    return out.reshape(16, 16)[:, 0]


# speedup vs baseline: 3.0151x; 1.0030x over previous
import functools

import jax
import jax.numpy as jnp
from jax import lax
from jax.experimental import pallas as pl
from jax.experimental.pallas import tpu as pltpu
from jax.experimental.pallas import tpu_sc as plsc

_K = 99074
_NSUB = 1
_UNROLL = 16
_HSZ = 2048 * 16 * _NSUB
_CHROWS = 32
_CHW = _CHROWS * 512
_HALF = 384 * 512
_NCH = 384 // _CHROWS
_PASSES = ((20, 11, -1), (10, 10, 20), (0, 10, 10))


def _haar_bits_kernel(x_ref, sign_ref, out_ref):
    v = x_ref[0, 0]
    z = v * sign_ref[...]
    t = z + jnp.concatenate([z[:, -1:], z[:, :-1]], axis=1)
    u = t + jnp.concatenate([t[-1:, :], t[:-1, :]], axis=0)
    ue = u.reshape(256, 2, 512)[:, 0, :]
    out_ref[0, 0] = jax.lax.bitcast_convert_type(jnp.abs(ue), jnp.int32)


def _sc_select(bits_hbm, sp_hbm, out_hbm, xh_hbm, msg_hbm,
               bufa, bufb, part, hist, msg, betav, spv, sema, semb):
    c = lax.axis_index("c")
    s = lax.axis_index("s")
    b = c * 8 + (s // 2)
    half = s % 2
    lane = lax.iota(jnp.int32, 16)
    w_gen = 1 - (lane & 1)

    resid = jnp.int32(_K)
    p_scal = jnp.int32(0)
    p_vec = jnp.zeros((16,), jnp.int32)

    bufs = (bufa, bufb)
    sems = (sema, semb)

    def start(ch, h):
        g = h * 384 + ch * _CHROWS
        ci, r0 = divmod(g, 256)
        i = ch & 1
        return pltpu.async_copy(
            bits_hbm.at[b, ci, pl.ds(r0, _CHROWS), :], bufs[i], sems[i])

    for shift, nbits, tshift in _PASSES:
        nbins = 1 << nbits
        hwords = nbins * 16

        @plsc.parallel_loop(0, _NSUB * nbins, step=1, unroll=_UNROLL)
        def _(i):
            hist[pl.ds(i * 16, 16)] = jnp.zeros((16,), jnp.int32)

        pend = [start(0, half)]
        for ch in range(_NCH):
            pend.pop().wait()
            if ch + 1 < _NCH:
                pend.append(start(ch + 1, half))
            buf = bufs[ch & 1]
            base_row = half * 384 + ch * _CHROWS

            @plsc.parallel_loop(0, _CHW // 16, step=1, unroll=_UNROLL)
            def _(i):
                v = buf[i >> 5, pl.ds((i & 31) * 16, 16)]
                bn = (v >> shift) & (nbins - 1)
                wv = w_gen + jnp.where(((i & 31) == 0) & (lane == 0), 1, 0)
                rfac = 1 + jnp.where((base_row + (i >> 5)) % 256 == 0, 1, 0)
                wv = wv * rfac
                if tshift >= 0:
                    wv = jnp.where((v >> tshift) == p_vec, wv, 0)
                plsc.addupdate_scatter(hist, [(bn << 4) + lane], wv)

        @pl.when(half == 1)
        def _():
            pltpu.sync_copy(hist.at[pl.ds(0, hwords)],
                            xh_hbm.at[pl.ds(b * hwords, hwords)])
        plsc.subcore_barrier()

        @pl.when(half == 0)
        def _():
            pltpu.sync_copy(xh_hbm.at[pl.ds(b * hwords, hwords)],
                            part.at[pl.ds(0, hwords)])

            @plsc.parallel_loop(0, nbins, step=1, unroll=_UNROLL)
            def _(i):
                blk = hist[pl.ds(i * 16, 16)] + part[pl.ds(i * 16, 16)]
                hist[pl.ds(i * 16, 16)] = blk

        ngroups = nbins // 16

        def group_body(i, carry):
            g = ngroups - 1 - i
            racc, gstar, above = carry
            acc = jnp.zeros((16,), jnp.int32)
            for k in range(16):
                acc = acc + hist[pl.ds(g * 256 + k * 16, 16)]
            t = jnp.sum(acc)
            found = (gstar < 0) & (racc + t >= resid)
            gstar = jnp.where(found, g, gstar)
            above = jnp.where(found, racc, above)
            return racc + t, gstar, above
        _, gstar, g_above = lax.fori_loop(
            0, ngroups, group_body,
            (jnp.int32(0), jnp.int32(-1), jnp.int32(0)))

        def bin_body(i, carry):
            bb = 15 - i
            bacc, bstar, above = carry
            t = jnp.sum(hist[pl.ds((gstar * 16 + bb) * 16, 16)])
            found = (bstar < 0) & (bacc + t >= resid)
            bstar = jnp.where(found, bb, bstar)
            above = jnp.where(found, bacc, above)
            return bacc + t, bstar, above
        _, bstar, b_above = lax.fori_loop(
            0, 16, bin_body, (g_above, jnp.int32(-1), jnp.int32(0)))

        bin_global = gstar * 16 + bstar
        resid = resid - b_above
        p_new = (p_scal << nbits) | bin_global

        @pl.when(half == 0)
        def _():
            msg[...] = jnp.full((16,), p_new, jnp.int32)
            pltpu.sync_copy(msg, msg_hbm.at[pl.ds(b * 16, 16)])
        plsc.subcore_barrier()
        pltpu.sync_copy(msg_hbm.at[pl.ds(b * 16, 16)], msg)
        p_vec = msg[...]
        p_scal = jnp.sum(jnp.where(lane == 0, p_vec, 0))

    med = lax.bitcast_convert_type(p_vec, jnp.float32)
    std = med / 0.6745
    pltpu.sync_copy(sp_hbm.at[pl.ds(0, 16)], spv.at[pl.ds(0, 16)])
    pltpu.sync_copy(sp_hbm.at[pl.ds(16, 16)], spv.at[pl.ds(16, 16)])
    spa = spv[pl.ds(0, 16)]
    spb = spv[pl.ds(16, 16)]
    g = spa * std + spb
    beta = 1.0 / (g * g)

    @pl.when(half == 0)
    def _():
        betav[...] = beta
        pltpu.sync_copy(betav, out_hbm.at[pl.ds(b * 16, 16)])


def kernel(x, a_k, b_k):
    x = x.astype(jnp.float32)

    rp = jax.lax.broadcasted_iota(jnp.int32, (512, 512), 0)
    cp = jax.lax.broadcasted_iota(jnp.int32, (512, 512), 1)
    sign = jnp.where(((rp ^ cp) & 1) == 0, jnp.float32(0.5), -0.5)
    bits = pl.pallas_call(
        _haar_bits_kernel,
        grid=(16, 3),
        in_specs=[
            pl.BlockSpec((1, 1, 512, 512), lambda i, j: (i, j, 0, 0)),
            pl.BlockSpec((512, 512), lambda i, j: (0, 0)),
        ],
        out_specs=pl.BlockSpec((1, 1, 256, 512), lambda i, j: (i, j, 0, 0)),
        out_shape=jax.ShapeDtypeStruct((16, 3, 256, 512), jnp.int32),
    )(x, sign)

    sp = jnp.broadcast_to(
        jax.nn.softplus(jnp.stack([a_k, b_k]))[:, None], (2, 16)).reshape(32)

    mesh = plsc.VectorSubcoreMesh(core_axis_name="c", subcore_axis_name="s")
    sc = functools.partial(
        pl.kernel, mesh=mesh,
        out_type=(
            jax.ShapeDtypeStruct((256,), jnp.float32),
            jax.ShapeDtypeStruct((16 * 2048 * 16,), jnp.int32),
            jax.ShapeDtypeStruct((256,), jnp.int32),
        ),
        scratch_types=[
            pltpu.VMEM((_CHROWS, 512), jnp.int32),
            pltpu.VMEM((_CHROWS, 512), jnp.int32),
            pltpu.VMEM((2048 * 16,), jnp.int32),
            pltpu.VMEM((_HSZ,), jnp.int32),
            pltpu.VMEM((16,), jnp.int32),
            pltpu.VMEM((16,), jnp.float32),
            pltpu.VMEM((32,), jnp.float32),
            pltpu.SemaphoreType.DMA,
            pltpu.SemaphoreType.DMA,
        ],
        compiler_params=pltpu.CompilerParams(needs_layout_passes=False),
    )(_sc_select)
    out, _, _ = sc(bits, sp)
    return out.reshape(16, 16)[:, 0]


# speedup vs baseline: 3.3499x; 1.1110x over previous
import functools

import jax
import jax.numpy as jnp
from jax import lax
from jax.experimental import pallas as pl
from jax.experimental.pallas import tpu as pltpu
from jax.experimental.pallas import tpu_sc as plsc

_K = 99074
_NSUB = 1
_UNROLL = 16
_HSZ = 2048 * 16 * _NSUB
_CHROWS = 32
_CHW = _CHROWS * 512
_HALF = 384 * 512
_NCH = 384 // _CHROWS
_PASSES = ((20, 11, -1), (10, 10, 20), (0, 10, 10))


def _haar_bits_kernel(x_ref, sign_ref, out_ref):
    v = x_ref[0]
    z = v * sign_ref[...]
    t = z + jnp.concatenate([z[:, :, -1:], z[:, :, :-1]], axis=2)
    u = t + jnp.concatenate([t[:, -1:, :], t[:, :-1, :]], axis=1)
    ue = u.reshape(3, 256, 2, 512)[:, :, 0, :]
    out_ref[0] = jax.lax.bitcast_convert_type(jnp.abs(ue), jnp.int32)


def _sc_select(bits_hbm, sp_hbm, out_hbm, xh_hbm, msg_hbm,
               bufa, bufb, part, hist, msg, betav, spv, sema, semb):
    c = lax.axis_index("c")
    s = lax.axis_index("s")
    b = c * 8 + (s // 2)
    half = s % 2
    lane = lax.iota(jnp.int32, 16)
    w_gen = 1 - (lane & 1)

    resid = jnp.int32(_K)
    p_scal = jnp.int32(0)
    p_vec = jnp.zeros((16,), jnp.int32)

    bufs = (bufa, bufb)
    sems = (sema, semb)

    def start(ch, h):
        g = h * 384 + ch * _CHROWS
        ci, r0 = divmod(g, 256)
        i = ch & 1
        return pltpu.async_copy(
            bits_hbm.at[b, ci, pl.ds(r0, _CHROWS), :], bufs[i], sems[i])

    for shift, nbits, tshift in _PASSES:
        nbins = 1 << nbits
        hwords = nbins * 16

        @plsc.parallel_loop(0, _NSUB * nbins, step=1, unroll=_UNROLL)
        def _(i):
            hist[pl.ds(i * 16, 16)] = jnp.zeros((16,), jnp.int32)

        pend = [start(0, half)]
        for ch in range(_NCH):
            pend.pop().wait()
            if ch + 1 < _NCH:
                pend.append(start(ch + 1, half))
            buf = bufs[ch & 1]
            base_row = half * 384 + ch * _CHROWS

            @plsc.parallel_loop(0, _CHW // 16, step=1, unroll=_UNROLL)
            def _(i):
                v = buf[i >> 5, pl.ds((i & 31) * 16, 16)]
                bn = (v >> shift) & (nbins - 1)
                wv = w_gen + jnp.where(((i & 31) == 0) & (lane == 0), 1, 0)
                rfac = 1 + jnp.where((base_row + (i >> 5)) % 256 == 0, 1, 0)
                wv = wv * rfac
                if tshift >= 0:
                    wv = jnp.where((v >> tshift) == p_vec, wv, 0)
                plsc.addupdate_scatter(hist, [(bn << 4) + lane], wv)

        @pl.when(half == 1)
        def _():
            pltpu.sync_copy(hist.at[pl.ds(0, hwords)],
                            xh_hbm.at[pl.ds(b * hwords, hwords)])
        plsc.subcore_barrier()

        @pl.when(half == 0)
        def _():
            pltpu.sync_copy(xh_hbm.at[pl.ds(b * hwords, hwords)],
                            part.at[pl.ds(0, hwords)])

            @plsc.parallel_loop(0, nbins, step=1, unroll=_UNROLL)
            def _(i):
                blk = hist[pl.ds(i * 16, 16)] + part[pl.ds(i * 16, 16)]
                hist[pl.ds(i * 16, 16)] = blk

        ngroups = nbins // 16

        def group_body(i, carry):
            g = ngroups - 1 - i
            racc, gstar, above = carry
            acc = jnp.zeros((16,), jnp.int32)
            for k in range(16):
                acc = acc + hist[pl.ds(g * 256 + k * 16, 16)]
            t = jnp.sum(acc)
            found = (gstar < 0) & (racc + t >= resid)
            gstar = jnp.where(found, g, gstar)
            above = jnp.where(found, racc, above)
            return racc + t, gstar, above
        _, gstar, g_above = lax.fori_loop(
            0, ngroups, group_body,
            (jnp.int32(0), jnp.int32(-1), jnp.int32(0)))

        def bin_body(i, carry):
            bb = 15 - i
            bacc, bstar, above = carry
            t = jnp.sum(hist[pl.ds((gstar * 16 + bb) * 16, 16)])
            found = (bstar < 0) & (bacc + t >= resid)
            bstar = jnp.where(found, bb, bstar)
            above = jnp.where(found, bacc, above)
            return bacc + t, bstar, above
        _, bstar, b_above = lax.fori_loop(
            0, 16, bin_body, (g_above, jnp.int32(-1), jnp.int32(0)))

        bin_global = gstar * 16 + bstar
        resid = resid - b_above
        p_new = (p_scal << nbits) | bin_global

        @pl.when(half == 0)
        def _():
            msg[...] = jnp.full((16,), p_new, jnp.int32)
            pltpu.sync_copy(msg, msg_hbm.at[pl.ds(b * 16, 16)])
        plsc.subcore_barrier()
        pltpu.sync_copy(msg_hbm.at[pl.ds(b * 16, 16)], msg)
        p_vec = msg[...]
        p_scal = jnp.sum(jnp.where(lane == 0, p_vec, 0))

    med = lax.bitcast_convert_type(p_vec, jnp.float32)
    std = med / 0.6745
    pltpu.sync_copy(sp_hbm.at[pl.ds(0, 16)], spv.at[pl.ds(0, 16)])
    pltpu.sync_copy(sp_hbm.at[pl.ds(16, 16)], spv.at[pl.ds(16, 16)])
    spa = spv[pl.ds(0, 16)]
    spb = spv[pl.ds(16, 16)]
    g = spa * std + spb
    beta = 1.0 / (g * g)

    @pl.when(half == 0)
    def _():
        betav[...] = beta
        pltpu.sync_copy(betav, out_hbm.at[pl.ds(b * 16, 16)])


def kernel(x, a_k, b_k):
    x = x.astype(jnp.float32)

    rp = jax.lax.broadcasted_iota(jnp.int32, (512, 512), 0)
    cp = jax.lax.broadcasted_iota(jnp.int32, (512, 512), 1)
    sign = jnp.where(((rp ^ cp) & 1) == 0, jnp.float32(0.5), -0.5)
    bits = pl.pallas_call(
        _haar_bits_kernel,
        grid=(16,),
        in_specs=[
            pl.BlockSpec((1, 3, 512, 512), lambda i: (i, 0, 0, 0)),
            pl.BlockSpec((512, 512), lambda i: (0, 0)),
        ],
        out_specs=pl.BlockSpec((1, 3, 256, 512), lambda i: (i, 0, 0, 0)),
        out_shape=jax.ShapeDtypeStruct((16, 3, 256, 512), jnp.int32),
    )(x, sign)

    sp = jnp.broadcast_to(
        jax.nn.softplus(jnp.stack([a_k, b_k]))[:, None], (2, 16)).reshape(32)

    mesh = plsc.VectorSubcoreMesh(core_axis_name="c", subcore_axis_name="s")
    sc = functools.partial(
        pl.kernel, mesh=mesh,
        out_type=(
            jax.ShapeDtypeStruct((256,), jnp.float32),
            jax.ShapeDtypeStruct((16 * 2048 * 16,), jnp.int32),
            jax.ShapeDtypeStruct((256,), jnp.int32),
        ),
        scratch_types=[
            pltpu.VMEM((_CHROWS, 512), jnp.int32),
            pltpu.VMEM((_CHROWS, 512), jnp.int32),
            pltpu.VMEM((2048 * 16,), jnp.int32),
            pltpu.VMEM((_HSZ,), jnp.int32),
            pltpu.VMEM((16,), jnp.int32),
            pltpu.VMEM((16,), jnp.float32),
            pltpu.VMEM((32,), jnp.float32),
            pltpu.SemaphoreType.DMA,
            pltpu.SemaphoreType.DMA,
        ],
        compiler_params=pltpu.CompilerParams(needs_layout_passes=False),
    )(_sc_select)
    out, _, _ = sc(bits, sp)
    return out.reshape(16, 16)[:, 0]
